# baseline (speedup 1.0000x reference)
import jax
import jax.numpy as jnp
from jax.experimental import pallas as pl
from jax.experimental.pallas import tpu as pltpu

N_TOK = 16 * 2048
D = 128
M = 1000
MP = 1024
TILE = 4096
T = N_TOK // TILE


def _body(q_ref, k_ref, out_ref, kb_ref, colmax_ref, acc_ref):
    t = pl.program_id(0)

    @pl.when(t == 0)
    def _init():
        colmax_ref[...] = jnp.full((1, MP), -1e30, jnp.float32)
        acc_ref[...] = jnp.zeros((D, MP), jnp.float32)
        kb_ref[...] = k_ref[...].astype(jnp.bfloat16)

    q = q_ref[...]
    ones = jnp.ones((D, D), jnp.bfloat16)
    ss = jnp.dot((q * q).astype(jnp.bfloat16), ones, preferred_element_type=jnp.float32)
    qn = q * jax.lax.rsqrt(jnp.maximum(ss, 1e-24))
    qb = qn.astype(jnp.bfloat16)
    s = jnp.dot(qb, kb_ref[...].T, preferred_element_type=jnp.float32)
    sb = s.astype(jnp.bfloat16)
    rowmax = jnp.max(sb, axis=1, keepdims=True)
    colmax_ref[...] = jnp.maximum(
        colmax_ref[...], jnp.max(sb, axis=0, keepdims=True).astype(jnp.float32)
    )

    onehot = jnp.where(sb == rowmax, jnp.bfloat16(1), jnp.bfloat16(0))
    qw = (qn * jnp.exp(rowmax.astype(jnp.float32))).astype(jnp.bfloat16)
    acc_ref[...] += jax.lax.dot_general(
        qw, onehot, (((0,), (0,)), ((), ())), preferred_element_type=jnp.float32
    )

    @pl.when(t == T - 1)
    def _finish():
        ut = 1e-05 * jnp.exp(-colmax_ref[...]) * acc_ref[...]
        upd = jnp.transpose(ut) + k_ref[...]
        nrm = jnp.sum(upd * upd, axis=1, keepdims=True)
        out_ref[...] = (upd * jax.lax.rsqrt(jnp.maximum(nrm, 1e-24)))[:M]


@jax.jit
def kernel(query, keys):
    q2 = query.reshape(N_TOK, D)
    kp = jnp.pad(keys, ((0, MP - M), (0, 0)))
    return pl.pallas_call(
        _body,
        grid=(T,),
        in_specs=[
            pl.BlockSpec((TILE, D), lambda t: (t, 0)),
            pl.BlockSpec((MP, D), lambda t: (0, 0)),
        ],
        out_specs=pl.BlockSpec((M, D), lambda t: (0, 0)),
        out_shape=jax.ShapeDtypeStruct((M, D), jnp.float32),
        scratch_shapes=[
            pltpu.VMEM((MP, D), jnp.bfloat16),
            pltpu.VMEM((1, MP), jnp.float32),
            pltpu.VMEM((D, MP), jnp.float32),
        ],
    )(q2, kp)
